# baseline (speedup 1.0000x reference)
import functools

import jax
import jax.numpy as jnp
from jax import lax
from jax.experimental import pallas as pl
from jax.experimental.pallas import tpu as pltpu
from jax.experimental.pallas import tpu_sc as plsc

C = 256
LANES = 16


@functools.lru_cache(maxsize=None)
def _build(batch: int, seq: int, V: int, D: int):
    mesh = plsc.VectorSubcoreMesh(core_axis_name="c", subcore_axis_name="s")
    nw = mesh.num_cores * mesh.num_subcores
    blk_per_s = batch // C
    bt_per_blk = C // 128
    dt = D // 8
    n_tasks = seq * blk_per_s
    assert n_tasks % nw == 0
    n = n_tasks // nw
    assert n >= 4 and n % 2 == 0

    @functools.partial(
        pl.kernel,
        out_type=jax.ShapeDtypeStruct((seq, dt, batch // 128, 8, 128),
                                      jnp.float32),
        mesh=mesh,
        scratch_types=[
            pltpu.VMEM((2, C), jnp.int32),
            pltpu.VMEM((2, C, D), jnp.float32),
            pltpu.VMEM((2, dt, bt_per_blk, 8, 128), jnp.float32),
            [pltpu.SemaphoreType.DMA] * 2,
            [pltpu.SemaphoreType.DMA] * 2,
            [pltpu.SemaphoreType.DMA] * 2,
        ],
        compiler_params=pltpu.CompilerParams(use_tc_tiling_on_sc=False,
                                             needs_layout_passes=False),
    )
    def k(idx_hbm, table_hbm, p4_hbm, ibuf, gbuf, tbuf, isems, gsems, ssems):
        wid = lax.axis_index("s") * mesh.num_cores + lax.axis_index("c")
        iota_c = lax.iota(jnp.int32, 16)

        def task_sb(j):
            tg = wid + nw * j
            s = tg // blk_per_s
            return s, tg - s * blk_per_s

        def start_idx(j, p):
            s, blk = task_sb(j)
            pltpu.async_copy(idx_hbm.at[pl.ds(s * batch + blk * C, C)],
                             ibuf.at[p], isems[p])

        def wait_idx(p):
            pltpu.make_async_copy(idx_hbm.at[pl.ds(0, C)], ibuf.at[p],
                                  isems[p]).wait()

        def start_gather(p):
            pltpu.async_copy(table_hbm.at[ibuf.at[p]], gbuf.at[p], gsems[p])

        def wait_gather(p):
            pltpu.make_async_copy(table_hbm.at[ibuf.at[p]], gbuf.at[p],
                                  gsems[p]).wait()

        def start_store(j, p):
            s, blk = task_sb(j)
            pltpu.async_copy(tbuf.at[p],
                             p4_hbm.at[s, :, pl.ds(blk * bt_per_blk,
                                                   bt_per_blk)],
                             ssems[p])

        def wait_store(p):
            pltpu.make_async_copy(tbuf.at[p],
                                  p4_hbm.at[0, :, pl.ds(0, bt_per_blk)],
                                  ssems[p]).wait()

        def transpose(p):
            g = gbuf.at[p]
            t = tbuf.at[p]

            @pl.loop(0, D)
            def _(i):
                di = i // 8
                dr = i - di * 8
                cvec = jnp.full((16,), i, jnp.int32)
                for btl in range(bt_per_blk):
                    for bcg in range(8):
                        ridx = iota_c + (btl * 128 + bcg * LANES)
                        vals = plsc.load_gather(g, [ridx, cvec])
                        t[di, btl, dr, pl.ds(bcg * LANES, LANES)] = vals

        def body(t, p, do_idx=True, do_gather=True, do_wait_store=True):
            wait_gather(p)
            if do_idx:
                start_idx(t + 2, p)
            if do_gather:
                wait_idx(1 - p)
                start_gather(1 - p)
            if do_wait_store:
                wait_store(p)
            transpose(p)
            start_store(t, p)

        start_idx(0, 0)
        start_idx(1, 1)
        wait_idx(0)
        start_gather(0)
        body(0, 0, do_wait_store=False)
        body(1, 1, do_wait_store=False)

        @pl.loop(1, n // 2 - 1)
        def _(u):
            for p01 in range(2):
                body(2 * u + p01, p01)

        body(n - 2, 0, do_idx=False)
        body(n - 1, 1, do_idx=False, do_gather=False)
        wait_store(0)
        wait_store(1)

    return k


def kernel(seqs, weight):
    batch, seq_len = seqs.shape
    vocab, d = weight.shape
    idx = seqs.T.reshape(-1).astype(jnp.int32)
    p4 = _build(batch, seq_len, vocab, d)(idx, weight)
    return (p4.transpose(2, 4, 0, 1, 3)
            .reshape(batch, seq_len, d))


# speedup vs baseline: 1.4579x; 1.4579x over previous
import functools

import jax
import jax.numpy as jnp
from jax import lax
from jax.experimental import pallas as pl
from jax.experimental.pallas import tpu as pltpu
from jax.experimental.pallas import tpu_sc as plsc

C = 256
LANES = 16


@functools.lru_cache(maxsize=None)
def _build(batch: int, seq: int, V: int, D: int):
    mesh = plsc.VectorSubcoreMesh(core_axis_name="c", subcore_axis_name="s")
    nw = mesh.num_cores * mesh.num_subcores
    blk_per_s = batch // C
    bt_per_blk = C // 128
    dt = D // 8
    n_tasks = seq * blk_per_s
    assert n_tasks % nw == 0
    n = n_tasks // nw
    assert n >= 4 and n % 2 == 0

    @functools.partial(
        pl.kernel,
        out_type=jax.ShapeDtypeStruct((seq, dt, batch // 128, 8, 128),
                                      jnp.float32),
        mesh=mesh,
        scratch_types=[
            pltpu.VMEM((2, C), jnp.int32),
            pltpu.VMEM((2, C, D), jnp.float32),
            pltpu.VMEM((2, dt, bt_per_blk, 8, 128), jnp.float32),
            [pltpu.SemaphoreType.DMA] * 2,
            [pltpu.SemaphoreType.DMA] * 2,
            [pltpu.SemaphoreType.DMA] * 2,
        ],
        compiler_params=pltpu.CompilerParams(use_tc_tiling_on_sc=False,
                                             needs_layout_passes=False),
    )
    def k(idx_hbm, table_hbm, p4_hbm, ibuf, gbuf, tbuf, isems, gsems, ssems):
        wid = lax.axis_index("s") * mesh.num_cores + lax.axis_index("c")
        iota_c = lax.iota(jnp.int32, 16)

        def task_sb(j):
            tg = wid + nw * j
            s = tg // blk_per_s
            return s, tg - s * blk_per_s

        def start_idx(j, p):
            s, blk = task_sb(j)
            pltpu.async_copy(idx_hbm.at[pl.ds(s * batch + blk * C, C)],
                             ibuf.at[p], isems[p])

        def wait_idx(p):
            pltpu.make_async_copy(idx_hbm.at[pl.ds(0, C)], ibuf.at[p],
                                  isems[p]).wait()

        def start_gather(p):
            pltpu.async_copy(table_hbm.at[ibuf.at[p]], gbuf.at[p], gsems[p])

        def wait_gather(p):
            pltpu.make_async_copy(table_hbm.at[ibuf.at[p]], gbuf.at[p],
                                  gsems[p]).wait()

        def start_store(j, p):
            s, blk = task_sb(j)
            pltpu.async_copy(tbuf.at[p],
                             p4_hbm.at[s, :, pl.ds(blk * bt_per_blk,
                                                   bt_per_blk)],
                             ssems[p])

        def wait_store(p):
            pltpu.make_async_copy(tbuf.at[p],
                                  p4_hbm.at[0, :, pl.ds(0, bt_per_blk)],
                                  ssems[p]).wait()

        def transpose(p):
            g = gbuf.at[p]
            t = tbuf.at[p]

            @plsc.parallel_loop(0, D)
            def _(i):
                di = i // 8
                dr = i - di * 8
                cvec = jnp.full((16,), i, jnp.int32)
                for btl in range(bt_per_blk):
                    for bcg in range(8):
                        ridx = iota_c + (btl * 128 + bcg * LANES)
                        vals = plsc.load_gather(g, [ridx, cvec])
                        t[di, btl, dr, pl.ds(bcg * LANES, LANES)] = vals

        def body(t, p, do_idx=True, do_gather=True, do_wait_store=True):
            wait_gather(p)
            if do_idx:
                start_idx(t + 2, p)
            if do_gather:
                wait_idx(1 - p)
                start_gather(1 - p)
            if do_wait_store:
                wait_store(p)
            transpose(p)
            start_store(t, p)

        start_idx(0, 0)
        start_idx(1, 1)
        wait_idx(0)
        start_gather(0)
        body(0, 0, do_wait_store=False)
        body(1, 1, do_wait_store=False)

        @pl.loop(1, n // 2 - 1)
        def _(u):
            for p01 in range(2):
                body(2 * u + p01, p01)

        body(n - 2, 0, do_idx=False)
        body(n - 1, 1, do_idx=False, do_gather=False)
        wait_store(0)
        wait_store(1)

    return k


def kernel(seqs, weight):
    batch, seq_len = seqs.shape
    vocab, d = weight.shape
    idx = seqs.T.reshape(-1).astype(jnp.int32)
    p4 = _build(batch, seq_len, vocab, d)(idx, weight)
    return (p4.transpose(2, 4, 0, 1, 3)
            .reshape(batch, seq_len, d))


# speedup vs baseline: 1.4765x; 1.0127x over previous
import functools

import jax
import jax.numpy as jnp
from jax import lax
from jax.experimental import pallas as pl
from jax.experimental.pallas import tpu as pltpu
from jax.experimental.pallas import tpu_sc as plsc

EMBED_DIM = 64
CHUNK = 256
NBUF = 4


@functools.lru_cache(maxsize=None)
def _build(B: int, V: int, D: int):
    mesh = plsc.VectorSubcoreMesh(core_axis_name="c", subcore_axis_name="s")
    nw = mesh.num_cores * mesh.num_subcores
    assert B % (nw * CHUNK) == 0
    b_per_w = B // nw
    n_chunks = b_per_w // CHUNK
    assert n_chunks % NBUF == 0 and n_chunks >= 3 * NBUF

    @functools.partial(
        pl.kernel,
        out_type=jax.ShapeDtypeStruct((B, D), jnp.float32),
        mesh=mesh,
        scratch_types=[
            pltpu.VMEM((b_per_w,), jnp.int32),
            pltpu.VMEM((NBUF, CHUNK, D), jnp.float32),
            [pltpu.SemaphoreType.DMA] * NBUF,
            [pltpu.SemaphoreType.DMA] * NBUF,
        ],
        compiler_params=pltpu.CompilerParams(use_tc_tiling_on_sc=False,
                                             needs_layout_passes=False),
    )
    def k(idx_hbm, table_hbm, out_hbm, idx_v, rows_v, gsems, ssems):
        wid = lax.axis_index("s") * mesh.num_cores + lax.axis_index("c")
        base = wid * b_per_w
        pltpu.sync_copy(idx_hbm.at[pl.ds(base, b_per_w)], idx_v)

        def start_gather(g, b):
            pltpu.async_copy(
                table_hbm.at[idx_v.at[pl.ds(g * CHUNK, CHUNK)]],
                rows_v.at[b], gsems[b])

        def wait_gather(b):
            pltpu.make_async_copy(
                table_hbm.at[idx_v.at[pl.ds(0, CHUNK)]],
                rows_v.at[b], gsems[b]).wait()

        def start_store(g, b):
            pltpu.async_copy(
                rows_v.at[b], out_hbm.at[pl.ds(base + g * CHUNK, CHUNK)],
                ssems[b])

        def wait_store(b):
            pltpu.make_async_copy(
                rows_v.at[b], out_hbm.at[pl.ds(base, CHUNK)], ssems[b]).wait()

        def body(g, b):
            b2 = (b + 2) % NBUF
            wait_gather(b)
            wait_store(b2)
            start_gather(g + 2, b2)
            start_store(g, b)

        start_gather(0, 0)
        start_gather(1, 1)
        for g in (0, 1):
            wait_gather(g)
            start_gather(g + 2, (g + 2) % NBUF)
            start_store(g, g)
        for g in (2, 3):
            body(g, g)

        @pl.loop(1, n_chunks // NBUF - 1)
        def _(t):
            for off in range(NBUF):
                body(t * NBUF + off, off)

        for g in range(n_chunks - NBUF, n_chunks):
            b = g % NBUF
            wait_gather(b)
            if g + 2 < n_chunks:
                wait_store((b + 2) % NBUF)
                start_gather(g + 2, (b + 2) % NBUF)
            start_store(g, b)
        for b in range(NBUF):
            wait_store(b)

    return k


def kernel(seqs, weight):
    batch, seq_len = seqs.shape
    vocab, d = weight.shape
    idx = seqs.reshape(-1).astype(jnp.int32)
    out = _build(batch * seq_len, vocab, d)(idx, weight)
    return out.reshape(batch, seq_len, d)


# speedup vs baseline: 2.3586x; 1.5974x over previous
import functools

import jax
import jax.numpy as jnp
from jax import lax
from jax.experimental import pallas as pl
from jax.experimental.pallas import tpu as pltpu
from jax.experimental.pallas import tpu_sc as plsc

C = 256
LANES = 16


@functools.lru_cache(maxsize=None)
def _build(batch: int, seq: int, V: int, D: int):
    mesh = plsc.VectorSubcoreMesh(core_axis_name="c", subcore_axis_name="s")
    nw = mesh.num_cores * mesh.num_subcores
    blk_per_s = batch // C
    bt_per_blk = C // 128
    dt = D // 8
    n_tasks = seq * blk_per_s
    assert n_tasks % nw == 0
    n = n_tasks // nw
    assert n >= 4 and n % 2 == 0

    @functools.partial(
        pl.kernel,
        out_type=jax.ShapeDtypeStruct((seq, dt, batch // 128, 8, 128),
                                      jnp.float32),
        mesh=mesh,
        scratch_types=[
            pltpu.VMEM((2, C), jnp.int32),
            pltpu.VMEM((2, C, D), jnp.float32),
            pltpu.VMEM((2, dt, bt_per_blk, 8, 129), jnp.float32),
            [pltpu.SemaphoreType.DMA] * 2,
            [pltpu.SemaphoreType.DMA] * 2,
            [pltpu.SemaphoreType.DMA] * 2,
        ],
        compiler_params=pltpu.CompilerParams(use_tc_tiling_on_sc=False,
                                             needs_layout_passes=False),
    )
    def k(idx_hbm, table_hbm, p4_hbm, ibuf, gbuf, tbuf, isems, gsems, ssems):
        wid = lax.axis_index("s") * mesh.num_cores + lax.axis_index("c")
        iota_c = lax.iota(jnp.int32, 16)

        def task_sb(j):
            tg = wid + nw * j
            s = tg // blk_per_s
            return s, tg - s * blk_per_s

        def start_idx(j, p):
            s, blk = task_sb(j)
            pltpu.async_copy(idx_hbm.at[pl.ds(s * batch + blk * C, C)],
                             ibuf.at[p], isems[p])

        def wait_idx(p):
            pltpu.make_async_copy(idx_hbm.at[pl.ds(0, C)], ibuf.at[p],
                                  isems[p]).wait()

        def start_gather(p):
            pltpu.async_copy(table_hbm.at[ibuf.at[p]], gbuf.at[p], gsems[p])

        def wait_gather(p):
            pltpu.make_async_copy(table_hbm.at[ibuf.at[p]], gbuf.at[p],
                                  gsems[p]).wait()

        def start_store(j, p):
            s, blk = task_sb(j)
            pltpu.async_copy(tbuf.at[p, :, :, :, pl.ds(0, 128)],
                             p4_hbm.at[s, :, pl.ds(blk * bt_per_blk,
                                                   bt_per_blk)],
                             ssems[p])

        def wait_store(p):
            pltpu.make_async_copy(tbuf.at[p, :, :, :, pl.ds(0, 128)],
                                  p4_hbm.at[0, :, pl.ds(0, bt_per_blk)],
                                  ssems[p]).wait()

        dtv = [(iota_c >= 8).astype(jnp.int32) + 2 * kk for kk in range(4)]
        drv_c = iota_c & 7

        def transpose(p):
            g = gbuf.at[p]
            t = tbuf.at[p]

            @plsc.parallel_loop(0, C, unroll=4)
            def _(b):
                btl = b // 128
                bc = b - btl * 128
                btl_v = jnp.full((16,), btl, jnp.int32)
                bc_v = jnp.full((16,), bc, jnp.int32)
                for kk in range(4):
                    vals = g[b, pl.ds(16 * kk, 16)]
                    plsc.store_scatter(t, [dtv[kk], btl_v, drv_c, bc_v],
                                       vals)

        def body(t, p, do_idx=True, do_gather=True, do_wait_store=True):
            wait_gather(p)
            if do_idx:
                start_idx(t + 2, p)
            if do_gather:
                wait_idx(1 - p)
                start_gather(1 - p)
            if do_wait_store:
                wait_store(p)
            transpose(p)
            start_store(t, p)

        start_idx(0, 0)
        start_idx(1, 1)
        wait_idx(0)
        start_gather(0)
        body(0, 0, do_wait_store=False)
        body(1, 1, do_wait_store=False)

        @pl.loop(1, n // 2 - 1)
        def _(u):
            for p01 in range(2):
                body(2 * u + p01, p01)

        body(n - 2, 0, do_idx=False)
        body(n - 1, 1, do_idx=False, do_gather=False)
        wait_store(0)
        wait_store(1)

    return k


def kernel(seqs, weight):
    batch, seq_len = seqs.shape
    vocab, d = weight.shape
    idx = seqs.T.reshape(-1).astype(jnp.int32)
    p4 = _build(batch, seq_len, vocab, d)(idx, weight)
    return (p4.transpose(2, 4, 0, 1, 3)
            .reshape(batch, seq_len, d))
